# baseline (speedup 1.0000x reference)
import functools

import jax
import jax.numpy as jnp
from jax import lax
from jax.experimental import pallas as pl
from jax.experimental.pallas import tpu as pltpu
from jax.experimental.pallas import tpu_sc as plsc

B, L, D = 1024, 200, 64
NC, NS = 2, 16
NW = NC * NS
BT = 128
NBT = B // BT
UNITS = L * NBT
UPW = UNITS // NW
LANES = 16
BG = BT // LANES
DT = D // 8


def _body(tok_hbm, idx_hbm, pos_hbm, out_hbm,
          idx_v, pos_v, g0, g1, o0, o1, gs0, gs1, ws0, ws1):
    wid = lax.axis_index("s") * NC + lax.axis_index("c")
    base_u = wid * UPW

    pltpu.sync_copy(idx_hbm.at[wid], idx_v.at[pl.ds(0, UPW)])
    pltpu.sync_copy(pos_hbm, pos_v)

    gbuf = (g0, g1)
    obuf = (o0, o1)
    gsem = (gs0, gs1)
    wsem = (ws0, ws1)

    for k in range(BT // LANES):
        idx_v[UPW, pl.ds(k * LANES, LANES)] = jnp.zeros((LANES,), jnp.int32)
        idx_v[UPW + 1, pl.ds(k * LANES, LANES)] = jnp.zeros((LANES,), jnp.int32)

    iota = lax.iota(jnp.int32, LANES)
    rowsel = [iota + (bg * LANES) for bg in range(BG)]

    def fire_gather(i, p):
        pltpu.async_copy(tok_hbm.at[idx_v.at[i]], gbuf[p], gsem[p])

    def wait_gather(p):
        pltpu.make_async_copy(tok_hbm.at[idx_v.at[0]], gbuf[p], gsem[p]).wait()

    def fire_writes(p, l, bt):
        for dt in range(DT):
            pltpu.async_copy(obuf[p].at[pl.ds(dt * 8, 8)],
                             out_hbm.at[l, dt, bt], wsem[p])

    def drain_writes(p):
        for dt in range(DT):
            pltpu.make_async_copy(obuf[p].at[pl.ds(dt * 8, 8)],
                                  out_hbm.at[0, dt, 0], wsem[p]).wait()

    def compute(p, u):
        ls = lax.rem(u, 8)
        bt = lax.rem(u // 8, NBT)
        l = (u // (8 * NBT)) * 8 + ls
        g = gbuf[p]
        o = obuf[p]

        def d_body(d, carry):
            dsel = jnp.full((LANES,), d, jnp.int32)
            lsel = jnp.full((LANES,), l, jnp.int32)
            padd = plsc.load_gather(pos_v, [lsel, dsel])
            for bg in range(BG):
                v = plsc.load_gather(g, [rowsel[bg], dsel])
                o[d, pl.ds(bg * LANES, LANES)] = v + padd
            return carry

        lax.fori_loop(0, D, d_body, 0, unroll=2)
        fire_writes(p, l, bt)

    fire_gather(0, 0)
    fire_gather(1, 1)
    wait_gather(0)
    compute(0, base_u)
    fire_gather(2, 0)
    wait_gather(1)
    compute(1, base_u + 1)

    def j_body(j, carry):
        a = 2 * j
        fire_gather(a + 1, 1)
        wait_gather(0)
        drain_writes(0)
        compute(0, base_u + a)
        fire_gather(a + 2, 0)
        wait_gather(1)
        drain_writes(1)
        compute(1, base_u + a + 1)
        return carry

    lax.fori_loop(1, UPW // 2, j_body, 0)

    wait_gather(0)
    drain_writes(0)
    drain_writes(1)


@jax.jit
def _embed(tok_table, idx3, pos):
    run = pl.kernel(
        _body,
        mesh=plsc.VectorSubcoreMesh(core_axis_name="c", subcore_axis_name="s"),
        compiler_params=pltpu.CompilerParams(
            use_tc_tiling_on_sc=False, needs_layout_passes=False),
        out_type=jax.ShapeDtypeStruct((L, DT, NBT, 8, BT), jnp.float32),
        scratch_types=[
            pltpu.VMEM((UPW + 2, BT), jnp.int32),
            pltpu.VMEM((L, D), jnp.float32),
            pltpu.VMEM((BT, D), jnp.float32),
            pltpu.VMEM((BT, D), jnp.float32),
            pltpu.VMEM((D, BT), jnp.float32),
            pltpu.VMEM((D, BT), jnp.float32),
            pltpu.SemaphoreType.DMA,
            pltpu.SemaphoreType.DMA,
            pltpu.SemaphoreType.DMA,
            pltpu.SemaphoreType.DMA,
        ],
    )
    return run(tok_table, idx3, pos)


def kernel(x, tok_table, pos_table):
    xT = jnp.swapaxes(x.astype(jnp.int32), 0, 1)
    idx3 = xT.reshape(L // 8, 8, NBT, BT).transpose(0, 2, 1, 3).reshape(NW, UPW, BT)
    pos = pos_table[:L]
    out5 = _embed(tok_table, idx3, pos)
    return out5.transpose(2, 4, 0, 1, 3).reshape(B, L, D)


# speedup vs baseline: 1.1699x; 1.1699x over previous
import functools

import jax
import jax.numpy as jnp
from jax import lax
from jax.experimental import pallas as pl
from jax.experimental.pallas import tpu as pltpu
from jax.experimental.pallas import tpu_sc as plsc

B, L, D = 1024, 200, 64
NC, NS = 2, 16
NW = NC * NS
BT = 128
NBT = B // BT
UNITS = L * NBT
UPW = UNITS // NW
LANES = 16
BG = BT // LANES
DT = D // 8


def _body(tok_hbm, idx_hbm, pos_hbm, out_hbm,
          idx_v, pos_v, g0, g1, o0, o1, gs0, gs1, ws0, ws1):
    wid = lax.axis_index("s") * NC + lax.axis_index("c")
    base_u = wid * UPW

    gbuf = (g0, g1)
    obuf = (o0, o1)
    gsem = (gs0, gs1)
    wsem = (ws0, ws1)

    iota = lax.iota(jnp.int32, LANES)
    rowsel = [iota + (bg * LANES) for bg in range(BG)]

    pltpu.sync_copy(pos_hbm, pos_v)

    stage = []
    for c in range(UPW // LANES + 1):
        uvec = iota + (base_u + c * LANES)
        lvec = jnp.right_shift(uvec, 3)
        btv = jnp.bitwise_and(uvec, 7)
        row = jnp.bitwise_or(
            jnp.bitwise_or(jnp.left_shift(jnp.right_shift(lvec, 3), 6),
                           jnp.left_shift(btv, 3)),
            jnp.bitwise_and(lvec, 7))
        row = jnp.minimum(row, UNITS - 1)
        stage.append(pltpu.async_copy(
            idx_hbm.at[row], idx_v.at[pl.ds(c * LANES, LANES)], gs0))
    for h in stage:
        h.wait()

    def fire_gather(i, p):
        pltpu.async_copy(tok_hbm.at[idx_v.at[i]], gbuf[p], gsem[p])

    def wait_gather(p):
        pltpu.make_async_copy(tok_hbm.at[idx_v.at[0]], gbuf[p], gsem[p]).wait()

    def fire_writes(p, l, bt):
        for dt in range(DT):
            pltpu.async_copy(obuf[p].at[pl.ds(dt * 8, 8)],
                             out_hbm.at[l, dt, bt], wsem[p])

    def drain_writes(p):
        for dt in range(DT):
            pltpu.make_async_copy(obuf[p].at[pl.ds(dt * 8, 8)],
                                  out_hbm.at[0, dt, 0], wsem[p]).wait()

    def compute(p, u):
        l = lax.shift_right_logical(u, 3)
        bt = jnp.bitwise_and(u, NBT - 1)
        g = gbuf[p]
        o = obuf[p]

        def d_body(d, carry):
            dsel = jnp.full((LANES,), d, jnp.int32)
            lsel = jnp.full((LANES,), l, jnp.int32)
            padd = plsc.load_gather(pos_v, [lsel, dsel])
            for bg in range(BG):
                v = plsc.load_gather(g, [rowsel[bg], dsel])
                o[d, pl.ds(bg * LANES, LANES)] = v + padd
            return carry

        lax.fori_loop(0, D, d_body, 0, unroll=2)
        fire_writes(p, l, bt)

    fire_gather(0, 0)
    fire_gather(1, 1)
    wait_gather(0)
    compute(0, base_u)
    fire_gather(2, 0)
    wait_gather(1)
    compute(1, base_u + 1)

    def j_body(j, carry):
        a = 2 * j
        fire_gather(a + 1, 1)
        wait_gather(0)
        drain_writes(0)
        compute(0, base_u + a)
        fire_gather(a + 2, 0)
        wait_gather(1)
        drain_writes(1)
        compute(1, base_u + a + 1)
        return carry

    lax.fori_loop(1, UPW // 2, j_body, 0)

    wait_gather(0)
    drain_writes(0)
    drain_writes(1)


@jax.jit
def _embed(tok_table, idx3, pos):
    run = pl.kernel(
        _body,
        mesh=plsc.VectorSubcoreMesh(core_axis_name="c", subcore_axis_name="s"),
        compiler_params=pltpu.CompilerParams(
            use_tc_tiling_on_sc=False, needs_layout_passes=False),
        out_type=jax.ShapeDtypeStruct((L, DT, NBT, 8, BT), jnp.float32),
        scratch_types=[
            pltpu.VMEM((64, BT), jnp.int32),
            pltpu.VMEM((L, D), jnp.float32),
            pltpu.VMEM((BT, D), jnp.float32),
            pltpu.VMEM((BT, D), jnp.float32),
            pltpu.VMEM((D, BT), jnp.float32),
            pltpu.VMEM((D, BT), jnp.float32),
            pltpu.SemaphoreType.DMA,
            pltpu.SemaphoreType.DMA,
            pltpu.SemaphoreType.DMA,
            pltpu.SemaphoreType.DMA,
        ],
    )
    return run(tok_table, idx3, pos)


def kernel(x, tok_table, pos_table):
    xT = jnp.swapaxes(x.astype(jnp.int32), 0, 1)
    idx3 = xT.reshape(L // 8, 8, NBT, BT).transpose(0, 2, 1, 3).reshape(UNITS, BT)
    pos = pos_table[:L]
    out5 = _embed(tok_table, idx3, pos)
    return out5.transpose(2, 4, 0, 1, 3).reshape(B, L, D)
